# Initial kernel scaffold; baseline (speedup 1.0000x reference)
import jax
import jax.numpy as jnp
from jax.experimental import pallas as pl


def kernel(x, edge_index, W1, b1, W2, b2):
    raise NotImplementedError("write your pallas kernel here")


# speedup vs baseline: 154.8557x; 154.8557x over previous
import functools

import jax
import jax.numpy as jnp
from jax import lax
from jax.experimental import pallas as pl
from jax.experimental.pallas import tpu as pltpu
from jax.experimental.pallas import tpu_sc as plsc

CHUNK = 2000
NUM_CORES = 2
NUM_SUBCORES = 16
NUM_TILES = NUM_CORES * NUM_SUBCORES


def _mesh():
  return plsc.VectorSubcoreMesh(core_axis_name="c", subcore_axis_name="s")


def _make_deg_kernel(e_pad, n_pad):
  per_tile = e_pad // NUM_TILES
  iters = per_tile // CHUNK
  zch = n_pad // NUM_SUBCORES

  @functools.partial(
      pl.kernel,
      mesh=_mesh(),
      compiler_params=pltpu.CompilerParams(needs_layout_passes=False),
      out_type=jax.ShapeDtypeStruct((NUM_CORES, n_pad), jnp.float32),
      scratch_types=[
          pltpu.VMEM((CHUNK,), jnp.int32),
          pltpu.VMEM((CHUNK,), jnp.float32),
          pltpu.VMEM((zch,), jnp.float32),
          pltpu.VMEM_SHARED((n_pad,), jnp.float32),
      ],
  )
  def deg_kernel(dst_hbm, out_hbm, dst_v, ones_v, zeros_v, table):
    c = lax.axis_index("c")
    s = lax.axis_index("s")

    @pl.loop(0, zch // 16)
    def _(i):
      zeros_v[pl.ds(i * 16, 16)] = jnp.zeros((16,), jnp.float32)

    @pl.loop(0, CHUNK // 16)
    def _(i):
      ones_v[pl.ds(i * 16, 16)] = jnp.ones((16,), jnp.float32)

    pltpu.sync_copy(zeros_v, table.at[pl.ds(s * zch, zch)])
    plsc.subcore_barrier()

    wid = c * NUM_SUBCORES + s

    @pl.loop(0, iters)
    def _(it):
      base = wid * per_tile + it * CHUNK
      pltpu.sync_copy(dst_hbm.at[pl.ds(base, CHUNK)], dst_v)
      pltpu.sync_copy(ones_v, table.at[dst_v], add=True)

    plsc.subcore_barrier()

    @pl.when(s == 0)
    def _():
      pltpu.sync_copy(table, out_hbm.at[c])

  return deg_kernel


def _make_agg1_kernel(e_pad, n_pad):
  per_tile = e_pad // NUM_TILES
  iters = per_tile // CHUNK
  zch = n_pad // NUM_SUBCORES

  @functools.partial(
      pl.kernel,
      mesh=_mesh(),
      compiler_params=pltpu.CompilerParams(needs_layout_passes=False),
      out_type=jax.ShapeDtypeStruct((NUM_CORES, n_pad), jnp.float32),
      scratch_types=[
          pltpu.VMEM((CHUNK,), jnp.int32),
          pltpu.VMEM((CHUNK,), jnp.int32),
          pltpu.VMEM((CHUNK,), jnp.float32),
          pltpu.VMEM((zch,), jnp.float32),
          pltpu.VMEM((n_pad,), jnp.float32),
          pltpu.VMEM_SHARED((n_pad,), jnp.float32),
      ],
  )
  def agg1_kernel(src_hbm, dst_hbm, a_hbm, out_hbm,
                  src_v, dst_v, vals_v, zeros_v, a_v, table):
    c = lax.axis_index("c")
    s = lax.axis_index("s")

    pltpu.sync_copy(a_hbm, a_v)

    @pl.loop(0, zch // 16)
    def _(i):
      zeros_v[pl.ds(i * 16, 16)] = jnp.zeros((16,), jnp.float32)

    pltpu.sync_copy(zeros_v, table.at[pl.ds(s * zch, zch)])
    plsc.subcore_barrier()

    wid = c * NUM_SUBCORES + s

    @pl.loop(0, iters)
    def _(it):
      base = wid * per_tile + it * CHUNK
      pltpu.sync_copy(src_hbm.at[pl.ds(base, CHUNK)], src_v)
      pltpu.sync_copy(dst_hbm.at[pl.ds(base, CHUNK)], dst_v)

      @pl.loop(0, CHUNK // 16)
      def _(j):
        idx = src_v[pl.ds(j * 16, 16)]
        vals_v[pl.ds(j * 16, 16)] = plsc.load_gather(a_v, [idx])

      pltpu.sync_copy(vals_v, table.at[dst_v], add=True)

    plsc.subcore_barrier()

    @pl.when(s == 0)
    def _():
      pltpu.sync_copy(table, out_hbm.at[c])

  return agg1_kernel


def _make_agg2_kernel(e_pad, n_pad):
  n_owners = NUM_CORES * 8
  per_owner = e_pad // n_owners
  iters = per_owner // CHUNK
  zch = n_pad // NUM_SUBCORES

  @functools.partial(
      pl.kernel,
      mesh=_mesh(),
      compiler_params=pltpu.CompilerParams(needs_layout_passes=False),
      out_type=jax.ShapeDtypeStruct((NUM_CORES, 2, n_pad), jnp.float32),
      scratch_types=[
          pltpu.VMEM((CHUNK,), jnp.int32),
          pltpu.VMEM((CHUNK,), jnp.int32),
          pltpu.VMEM((CHUNK,), jnp.float32),
          pltpu.VMEM((zch,), jnp.float32),
          pltpu.VMEM((n_pad,), jnp.float32),
          pltpu.VMEM_SHARED((n_pad,), jnp.float32),
          pltpu.VMEM_SHARED((n_pad,), jnp.float32),
      ],
  )
  def agg2_kernel(src_hbm, dst_hbm, g0_hbm, g1_hbm, out_hbm,
                  src_v, dst_v, vals_v, zeros_v, g_v, table0, table1):
    c = lax.axis_index("c")
    s = lax.axis_index("s")
    feat = s // 8
    owner = c * 8 + (s % 8)

    @pl.when(feat == 0)
    def _():
      pltpu.sync_copy(g0_hbm, g_v)

    @pl.when(feat == 1)
    def _():
      pltpu.sync_copy(g1_hbm, g_v)

    @pl.loop(0, zch // 16)
    def _(i):
      zeros_v[pl.ds(i * 16, 16)] = jnp.zeros((16,), jnp.float32)

    pltpu.sync_copy(zeros_v, table0.at[pl.ds(s * zch, zch)])
    pltpu.sync_copy(zeros_v, table1.at[pl.ds(s * zch, zch)])
    plsc.subcore_barrier()

    @pl.loop(0, iters)
    def _(it):
      base = owner * per_owner + it * CHUNK
      pltpu.sync_copy(src_hbm.at[pl.ds(base, CHUNK)], src_v)
      pltpu.sync_copy(dst_hbm.at[pl.ds(base, CHUNK)], dst_v)

      @pl.loop(0, CHUNK // 16)
      def _(j):
        idx = src_v[pl.ds(j * 16, 16)]
        vals_v[pl.ds(j * 16, 16)] = plsc.load_gather(g_v, [idx])

      @pl.when(feat == 0)
      def _():
        pltpu.sync_copy(vals_v, table0.at[dst_v], add=True)

      @pl.when(feat == 1)
      def _():
        pltpu.sync_copy(vals_v, table1.at[dst_v], add=True)

    plsc.subcore_barrier()

    @pl.when(s == 0)
    def _():
      pltpu.sync_copy(table0, out_hbm.at[c, 0])
      pltpu.sync_copy(table1, out_hbm.at[c, 1])

  return agg2_kernel


def _tc1_body(degp_ref, x_ref, dinv_ref, a_ref):
  deg = degp_ref[0] + degp_ref[1] + 1.0
  dinv = lax.rsqrt(deg)
  dinv_ref[...] = dinv
  a_ref[...] = dinv * x_ref[...]


def _tc2_body(t1p_ref, a_ref, dinv_ref, w1_ref, b1_ref, w2_ref, b2_ref,
              g0_ref, g1_ref):
  del b2_ref
  dinv = dinv_ref[...]
  t1 = t1p_ref[0] + t1p_ref[1] + a_ref[...]
  sval = dinv * t1
  p0 = jnp.zeros_like(sval)
  p1 = jnp.zeros_like(sval)
  for j in range(32):
    hj = jnp.maximum(sval * w1_ref[0, j] + b1_ref[0, j], 0.0)
    p0 = p0 + hj * w2_ref[j, 0]
    p1 = p1 + hj * w2_ref[j, 1]
  g0_ref[...] = dinv * p0
  g1_ref[...] = dinv * p1


def _tc3_body(t2p_ref, g0_ref, g1_ref, dinv_ref, b2_ref, o0_ref, o1_ref):
  dinv = dinv_ref[...]
  z0 = dinv * (t2p_ref[0, 0] + t2p_ref[1, 0] + g0_ref[...]) + b2_ref[0, 0]
  z1 = dinv * (t2p_ref[0, 1] + t2p_ref[1, 1] + g1_ref[...]) + b2_ref[0, 1]
  m = jnp.maximum(z0, z1)
  lse = m + jnp.log(jnp.exp(z0 - m) + jnp.exp(z1 - m))
  o0_ref[...] = z0 - lse
  o1_ref[...] = z1 - lse


def kernel(x, edge_index, W1, b1, W2, b2):
  n = x.shape[0]
  e = edge_index.shape[1]

  n_pad = ((n + 1023) // 1024) * 1024
  rows = n_pad // 128
  e_unit = NUM_TILES * CHUNK
  e_pad = ((e + e_unit - 1) // e_unit) * e_unit

  src = edge_index[0]
  dst = edge_index[1]
  if e_pad != e:
    src = jnp.concatenate([src, jnp.full((e_pad - e,), n, jnp.int32)])
    dst = jnp.concatenate([dst, jnp.full((e_pad - e,), n, jnp.int32)])

  x_flat = jnp.pad(x[:, 0], (0, n_pad - n))

  deg_p = _make_deg_kernel(e_pad, n_pad)(dst)

  degp_r = deg_p.reshape(NUM_CORES, rows, 128)
  x_r = x_flat.reshape(rows, 128)
  dinv_r, a_r = pl.pallas_call(
      _tc1_body,
      out_shape=[
          jax.ShapeDtypeStruct((rows, 128), jnp.float32),
          jax.ShapeDtypeStruct((rows, 128), jnp.float32),
      ],
  )(degp_r, x_r)

  t1_p = _make_agg1_kernel(e_pad, n_pad)(src, dst, a_r.reshape(n_pad))

  t1p_r = t1_p.reshape(NUM_CORES, rows, 128)
  g0_r, g1_r = pl.pallas_call(
      _tc2_body,
      out_shape=[
          jax.ShapeDtypeStruct((rows, 128), jnp.float32),
          jax.ShapeDtypeStruct((rows, 128), jnp.float32),
      ],
  )(t1p_r, a_r, dinv_r, W1.reshape(1, 32), b1.reshape(1, 32),
    W2.reshape(32, 2), b2.reshape(1, 2))

  t2_p = _make_agg2_kernel(e_pad, n_pad)(
      src, dst, g0_r.reshape(n_pad), g1_r.reshape(n_pad))

  t2p_r = t2_p.reshape(NUM_CORES, 2, rows, 128)
  o0_r, o1_r = pl.pallas_call(
      _tc3_body,
      out_shape=[
          jax.ShapeDtypeStruct((rows, 128), jnp.float32),
          jax.ShapeDtypeStruct((rows, 128), jnp.float32),
      ],
  )(t2p_r, g0_r, g1_r, dinv_r, b2.reshape(1, 2))

  return jnp.stack([o0_r.reshape(n_pad)[:n], o1_r.reshape(n_pad)[:n]], axis=1)


# speedup vs baseline: 198.0120x; 1.2787x over previous
import functools

import jax
import jax.numpy as jnp
from jax import lax
from jax.experimental import pallas as pl
from jax.experimental.pallas import tpu as pltpu
from jax.experimental.pallas import tpu_sc as plsc

CHUNK = 2000
NUM_CORES = 2
NUM_SUBCORES = 16
NUM_TILES = NUM_CORES * NUM_SUBCORES


def _mesh():
  return plsc.VectorSubcoreMesh(core_axis_name="c", subcore_axis_name="s")


def _fill(ref, size, value):
  vec = jnp.full((16,), value, jnp.float32)

  @pl.loop(0, size // 16)
  def _(i):
    ref[pl.ds(i * 16, 16)] = vec


def _zero_my_slice(zeros_v, table, s, zch):
  pltpu.sync_copy(zeros_v, table.at[pl.ds(s * zch, zch)])


def _make_deg_kernel(e_pad, n_pad):
  per_tile = e_pad // NUM_TILES
  pairs = per_tile // CHUNK // 2
  zch = n_pad // NUM_SUBCORES

  @functools.partial(
      pl.kernel,
      mesh=_mesh(),
      compiler_params=pltpu.CompilerParams(needs_layout_passes=False),
      out_type=jax.ShapeDtypeStruct((NUM_CORES, n_pad), jnp.float32),
      scratch_types=[
          pltpu.VMEM((CHUNK,), jnp.int32),
          pltpu.VMEM((CHUNK,), jnp.int32),
          pltpu.VMEM((CHUNK,), jnp.float32),
          pltpu.VMEM((zch,), jnp.float32),
          pltpu.VMEM_SHARED((n_pad,), jnp.float32),
          pltpu.SemaphoreType.DMA,
          pltpu.SemaphoreType.DMA,
      ],
  )
  def deg_kernel(dst_hbm, out_hbm, dst0, dst1, ones_v, zeros_v, table,
                 sem0, sem1):
    c = lax.axis_index("c")
    s = lax.axis_index("s")
    _fill(zeros_v, zch, 0.0)
    _fill(ones_v, CHUNK, 1.0)
    _zero_my_slice(zeros_v, table, s, zch)
    plsc.subcore_barrier()

    base0 = (c * NUM_SUBCORES + s) * per_tile

    @pl.loop(0, pairs)
    def _(k):
      @pl.when(k > 0)
      def _():
        pltpu.make_async_copy(ones_v, table.at[dst0], sem0).wait()

      pltpu.sync_copy(dst_hbm.at[pl.ds(base0 + 2 * k * CHUNK, CHUNK)], dst0)
      pltpu.async_copy(ones_v, table.at[dst0], sem0, add=True)

      @pl.when(k > 0)
      def _():
        pltpu.make_async_copy(ones_v, table.at[dst1], sem1).wait()

      pltpu.sync_copy(
          dst_hbm.at[pl.ds(base0 + (2 * k + 1) * CHUNK, CHUNK)], dst1)
      pltpu.async_copy(ones_v, table.at[dst1], sem1, add=True)

    pltpu.make_async_copy(ones_v, table.at[dst0], sem0).wait()
    pltpu.make_async_copy(ones_v, table.at[dst1], sem1).wait()
    plsc.subcore_barrier()

    @pl.when(s == 0)
    def _():
      pltpu.sync_copy(table, out_hbm.at[c])

  return deg_kernel


def _make_agg1_kernel(e_pad, n_pad):
  per_tile = e_pad // NUM_TILES
  pairs = per_tile // CHUNK // 2
  zch = n_pad // NUM_SUBCORES

  @functools.partial(
      pl.kernel,
      mesh=_mesh(),
      compiler_params=pltpu.CompilerParams(needs_layout_passes=False),
      out_type=jax.ShapeDtypeStruct((NUM_CORES, n_pad), jnp.float32),
      scratch_types=[
          pltpu.VMEM((CHUNK,), jnp.int32),
          pltpu.VMEM((CHUNK,), jnp.int32),
          pltpu.VMEM((CHUNK,), jnp.int32),
          pltpu.VMEM((CHUNK,), jnp.int32),
          pltpu.VMEM((CHUNK,), jnp.float32),
          pltpu.VMEM((CHUNK,), jnp.float32),
          pltpu.VMEM((zch,), jnp.float32),
          pltpu.VMEM((n_pad,), jnp.float32),
          pltpu.VMEM_SHARED((n_pad,), jnp.float32),
          pltpu.SemaphoreType.DMA,
          pltpu.SemaphoreType.DMA,
      ],
  )
  def agg1_kernel(src_hbm, dst_hbm, a_hbm, out_hbm,
                  src0, src1, dst0, dst1, vals0, vals1, zeros_v, a_v, table,
                  sem0, sem1):
    c = lax.axis_index("c")
    s = lax.axis_index("s")

    pltpu.sync_copy(a_hbm, a_v)
    _fill(zeros_v, zch, 0.0)
    _zero_my_slice(zeros_v, table, s, zch)
    plsc.subcore_barrier()

    base0 = (c * NUM_SUBCORES + s) * per_tile

    def gather(src_v, vals_v):
      @pl.loop(0, CHUNK // 16)
      def _(j):
        idx = src_v[pl.ds(j * 16, 16)]
        vals_v[pl.ds(j * 16, 16)] = plsc.load_gather(a_v, [idx])

    @pl.loop(0, pairs)
    def _(k):
      @pl.when(k > 0)
      def _():
        pltpu.make_async_copy(vals0, table.at[dst0], sem0).wait()

      base = base0 + 2 * k * CHUNK
      pltpu.sync_copy(src_hbm.at[pl.ds(base, CHUNK)], src0)
      pltpu.sync_copy(dst_hbm.at[pl.ds(base, CHUNK)], dst0)
      gather(src0, vals0)
      pltpu.async_copy(vals0, table.at[dst0], sem0, add=True)

      @pl.when(k > 0)
      def _():
        pltpu.make_async_copy(vals1, table.at[dst1], sem1).wait()

      base = base0 + (2 * k + 1) * CHUNK
      pltpu.sync_copy(src_hbm.at[pl.ds(base, CHUNK)], src1)
      pltpu.sync_copy(dst_hbm.at[pl.ds(base, CHUNK)], dst1)
      gather(src1, vals1)
      pltpu.async_copy(vals1, table.at[dst1], sem1, add=True)

    pltpu.make_async_copy(vals0, table.at[dst0], sem0).wait()
    pltpu.make_async_copy(vals1, table.at[dst1], sem1).wait()
    plsc.subcore_barrier()

    @pl.when(s == 0)
    def _():
      pltpu.sync_copy(table, out_hbm.at[c])

  return agg1_kernel


def _make_agg2_kernel(e_pad, n_pad):
  per_tile = e_pad // NUM_TILES
  pairs = per_tile // CHUNK // 2
  zch = n_pad // NUM_SUBCORES

  @functools.partial(
      pl.kernel,
      mesh=_mesh(),
      compiler_params=pltpu.CompilerParams(needs_layout_passes=False),
      out_type=jax.ShapeDtypeStruct((NUM_CORES, 2, n_pad), jnp.float32),
      scratch_types=[
          pltpu.VMEM((CHUNK,), jnp.int32),
          pltpu.VMEM((CHUNK,), jnp.int32),
          pltpu.VMEM((CHUNK,), jnp.int32),
          pltpu.VMEM((CHUNK,), jnp.int32),
          pltpu.VMEM((CHUNK,), jnp.float32),
          pltpu.VMEM((CHUNK,), jnp.float32),
          pltpu.VMEM((zch,), jnp.float32),
          pltpu.VMEM((n_pad,), jnp.float32),
          pltpu.VMEM_SHARED((n_pad,), jnp.float32),
          pltpu.SemaphoreType.DMA,
          pltpu.SemaphoreType.DMA,
      ],
  )
  def agg2_kernel(src_hbm, dst_hbm, g0_hbm, g1_hbm, out_hbm,
                  src0, src1, dst0, dst1, vals0, vals1, zeros_v, g_v,
                  table, sem0, sem1):
    c = lax.axis_index("c")
    s = lax.axis_index("s")
    _fill(zeros_v, zch, 0.0)

    base0 = (c * NUM_SUBCORES + s) * per_tile

    def gather(src_v, vals_v):
      @pl.loop(0, CHUNK // 16)
      def _(j):
        idx = src_v[pl.ds(j * 16, 16)]
        vals_v[pl.ds(j * 16, 16)] = plsc.load_gather(g_v, [idx])

    for f, g_hbm in enumerate((g0_hbm, g1_hbm)):
      pltpu.sync_copy(g_hbm, g_v)
      _zero_my_slice(zeros_v, table, s, zch)
      plsc.subcore_barrier()

      @pl.loop(0, pairs)
      def _(k):
        @pl.when(k > 0)
        def _():
          pltpu.make_async_copy(vals0, table.at[dst0], sem0).wait()

        base = base0 + 2 * k * CHUNK
        pltpu.sync_copy(src_hbm.at[pl.ds(base, CHUNK)], src0)
        pltpu.sync_copy(dst_hbm.at[pl.ds(base, CHUNK)], dst0)
        gather(src0, vals0)
        pltpu.async_copy(vals0, table.at[dst0], sem0, add=True)

        @pl.when(k > 0)
        def _():
          pltpu.make_async_copy(vals1, table.at[dst1], sem1).wait()

        base = base0 + (2 * k + 1) * CHUNK
        pltpu.sync_copy(src_hbm.at[pl.ds(base, CHUNK)], src1)
        pltpu.sync_copy(dst_hbm.at[pl.ds(base, CHUNK)], dst1)
        gather(src1, vals1)
        pltpu.async_copy(vals1, table.at[dst1], sem1, add=True)

      pltpu.make_async_copy(vals0, table.at[dst0], sem0).wait()
      pltpu.make_async_copy(vals1, table.at[dst1], sem1).wait()
      plsc.subcore_barrier()

      @pl.when(s == 0)
      def _():
        pltpu.sync_copy(table, out_hbm.at[c, f])

      plsc.subcore_barrier()

  return agg2_kernel


def _tc1_body(degp_ref, x_ref, dinv_ref, a_ref):
  deg = degp_ref[0] + degp_ref[1] + 1.0
  dinv = lax.rsqrt(deg)
  dinv_ref[...] = dinv
  a_ref[...] = dinv * x_ref[...]


def _tc2_body(t1p_ref, a_ref, dinv_ref, w1_ref, b1_ref, w2_ref, b2_ref,
              g0_ref, g1_ref):
  del b2_ref
  dinv = dinv_ref[...]
  t1 = t1p_ref[0] + t1p_ref[1] + a_ref[...]
  sval = dinv * t1
  p0 = jnp.zeros_like(sval)
  p1 = jnp.zeros_like(sval)
  for j in range(32):
    hj = jnp.maximum(sval * w1_ref[0, j] + b1_ref[0, j], 0.0)
    p0 = p0 + hj * w2_ref[j, 0]
    p1 = p1 + hj * w2_ref[j, 1]
  g0_ref[...] = dinv * p0
  g1_ref[...] = dinv * p1


def _tc3_body(t2p_ref, g0_ref, g1_ref, dinv_ref, b2_ref, o0_ref, o1_ref):
  dinv = dinv_ref[...]
  z0 = dinv * (t2p_ref[0, 0] + t2p_ref[1, 0] + g0_ref[...]) + b2_ref[0, 0]
  z1 = dinv * (t2p_ref[0, 1] + t2p_ref[1, 1] + g1_ref[...]) + b2_ref[0, 1]
  m = jnp.maximum(z0, z1)
  lse = m + jnp.log(jnp.exp(z0 - m) + jnp.exp(z1 - m))
  o0_ref[...] = z0 - lse
  o1_ref[...] = z1 - lse


def kernel(x, edge_index, W1, b1, W2, b2):
  n = x.shape[0]
  e = edge_index.shape[1]

  n_pad = ((n + 1023) // 1024) * 1024
  rows = n_pad // 128
  e_unit = NUM_TILES * CHUNK * 2
  e_pad = ((e + e_unit - 1) // e_unit) * e_unit

  src = edge_index[0]
  dst = edge_index[1]
  if e_pad != e:
    src = jnp.concatenate([src, jnp.full((e_pad - e,), n, jnp.int32)])
    dst = jnp.concatenate([dst, jnp.full((e_pad - e,), n, jnp.int32)])

  x_flat = jnp.pad(x[:, 0], (0, n_pad - n))

  deg_p = _make_deg_kernel(e_pad, n_pad)(dst)

  degp_r = deg_p.reshape(NUM_CORES, rows, 128)
  x_r = x_flat.reshape(rows, 128)
  dinv_r, a_r = pl.pallas_call(
      _tc1_body,
      out_shape=[
          jax.ShapeDtypeStruct((rows, 128), jnp.float32),
          jax.ShapeDtypeStruct((rows, 128), jnp.float32),
      ],
  )(degp_r, x_r)

  t1_p = _make_agg1_kernel(e_pad, n_pad)(src, dst, a_r.reshape(n_pad))

  t1p_r = t1_p.reshape(NUM_CORES, rows, 128)
  g0_r, g1_r = pl.pallas_call(
      _tc2_body,
      out_shape=[
          jax.ShapeDtypeStruct((rows, 128), jnp.float32),
          jax.ShapeDtypeStruct((rows, 128), jnp.float32),
      ],
  )(t1p_r, a_r, dinv_r, W1.reshape(1, 32), b1.reshape(1, 32),
    W2.reshape(32, 2), b2.reshape(1, 2))

  t2_p = _make_agg2_kernel(e_pad, n_pad)(
      src, dst, g0_r.reshape(n_pad), g1_r.reshape(n_pad))

  t2p_r = t2_p.reshape(NUM_CORES, 2, rows, 128)
  o0_r, o1_r = pl.pallas_call(
      _tc3_body,
      out_shape=[
          jax.ShapeDtypeStruct((rows, 128), jnp.float32),
          jax.ShapeDtypeStruct((rows, 128), jnp.float32),
      ],
  )(t2p_r, g0_r, g1_r, dinv_r, b2.reshape(1, 2))

  return jnp.stack([o0_r.reshape(n_pad)[:n], o1_r.reshape(n_pad)[:n]], axis=1)


# speedup vs baseline: 295.6779x; 1.4932x over previous
import functools

import jax
import jax.numpy as jnp
from jax import lax
from jax.experimental import pallas as pl
from jax.experimental.pallas import tpu as pltpu
from jax.experimental.pallas import tpu_sc as plsc

CHUNK = 2000
NUM_CORES = 2
NUM_SUBCORES = 16
NUM_TILES = NUM_CORES * NUM_SUBCORES


def _mesh():
  return plsc.VectorSubcoreMesh(core_axis_name="c", subcore_axis_name="s")


def _fill(ref, size, value):
  vec = jnp.full((16,), value, jnp.float32)

  @pl.loop(0, size // 16)
  def _(i):
    ref[pl.ds(i * 16, 16)] = vec


def _zero_my_slice(zeros_v, table, s, zch):
  pltpu.sync_copy(zeros_v, table.at[pl.ds(s * zch, zch)])


def _zero_slice_via(buf, table, s, zch):
  off = 0
  while off < zch:
    ln = min(CHUNK, zch - off)
    pltpu.sync_copy(buf.at[pl.ds(0, ln)], table.at[pl.ds(s * zch + off, ln)])
    off += ln


def _make_deg_kernel(e_pad, n_pad):
  per_tile = e_pad // NUM_TILES
  pairs = per_tile // CHUNK // 2
  zch = n_pad // NUM_SUBCORES

  @functools.partial(
      pl.kernel,
      mesh=_mesh(),
      compiler_params=pltpu.CompilerParams(needs_layout_passes=False),
      out_type=jax.ShapeDtypeStruct((NUM_CORES, n_pad), jnp.float32),
      scratch_types=[
          pltpu.VMEM((CHUNK,), jnp.int32),
          pltpu.VMEM((CHUNK,), jnp.int32),
          pltpu.VMEM((CHUNK,), jnp.float32),
          pltpu.VMEM((zch,), jnp.float32),
          pltpu.VMEM_SHARED((n_pad,), jnp.float32),
          pltpu.SemaphoreType.DMA,
          pltpu.SemaphoreType.DMA,
      ],
  )
  def deg_kernel(dst_hbm, out_hbm, dst0, dst1, ones_v, zeros_v, table,
                 sem0, sem1):
    c = lax.axis_index("c")
    s = lax.axis_index("s")
    _fill(zeros_v, zch, 0.0)
    _fill(ones_v, CHUNK, 1.0)
    _zero_my_slice(zeros_v, table, s, zch)
    plsc.subcore_barrier()

    base0 = (c * NUM_SUBCORES + s) * per_tile

    @pl.loop(0, pairs)
    def _(k):
      @pl.when(k > 0)
      def _():
        pltpu.make_async_copy(ones_v, table.at[dst0], sem0).wait()

      pltpu.sync_copy(dst_hbm.at[pl.ds(base0 + 2 * k * CHUNK, CHUNK)], dst0)
      pltpu.async_copy(ones_v, table.at[dst0], sem0, add=True)

      @pl.when(k > 0)
      def _():
        pltpu.make_async_copy(ones_v, table.at[dst1], sem1).wait()

      pltpu.sync_copy(
          dst_hbm.at[pl.ds(base0 + (2 * k + 1) * CHUNK, CHUNK)], dst1)
      pltpu.async_copy(ones_v, table.at[dst1], sem1, add=True)

    pltpu.make_async_copy(ones_v, table.at[dst0], sem0).wait()
    pltpu.make_async_copy(ones_v, table.at[dst1], sem1).wait()
    plsc.subcore_barrier()

    @pl.when(s == 0)
    def _():
      pltpu.sync_copy(table, out_hbm.at[c])

  return deg_kernel


def _agg_pipeline(src_hbm, dst_hbm, tab_v, table, base0, nchunks,
                  srcs, dsts, vals, lsems, ssems):
  quads = nchunks // 4

  def gather(src_v, vals_v):
    @pl.loop(0, CHUNK // 16, unroll=4)
    def _(j):
      idx = src_v[pl.ds(j * 16, 16)]
      vals_v[pl.ds(j * 16, 16)] = plsc.load_gather(tab_v, [idx])

  def fire_loads(chunk, j):
    base = base0 + chunk * CHUNK
    pltpu.async_copy(src_hbm.at[pl.ds(base, CHUNK)], srcs[j], lsems[j])
    pltpu.async_copy(dst_hbm.at[pl.ds(base, CHUNK)], dsts[j], lsems[j])

  def wait_loads(j):
    pltpu.make_async_copy(src_hbm.at[pl.ds(0, CHUNK)], srcs[j],
                          lsems[j]).wait()
    pltpu.make_async_copy(dst_hbm.at[pl.ds(0, CHUNK)], dsts[j],
                          lsems[j]).wait()

  def wait_scat(j):
    pltpu.make_async_copy(vals[j], table.at[dsts[j]], ssems[j]).wait()

  fire_loads(0, 0)
  fire_loads(1, 1)

  @pl.loop(0, quads)
  def _(k):
    for j in range(4):
      jp = (j + 2) % 4
      wait_loads(j)
      if j < 2:
        @pl.when(k > 0)
        def _():
          wait_scat(jp)

        fire_loads(4 * k + j + 2, jp)
      else:
        wait_scat(jp)

        @pl.when(k < quads - 1)
        def _():
          fire_loads(4 * k + j + 2, jp)

      gather(srcs[j], vals[j])
      pltpu.async_copy(vals[j], table.at[dsts[j]], ssems[j], add=True)

  wait_scat(2)
  wait_scat(3)


def _make_agg1_kernel(e_pad, n_pad, tab_n):
  per_tile = e_pad // NUM_TILES
  nchunks = per_tile // CHUNK
  zch = tab_n // NUM_SUBCORES

  @functools.partial(
      pl.kernel,
      mesh=_mesh(),
      compiler_params=pltpu.CompilerParams(needs_layout_passes=False),
      out_type=jax.ShapeDtypeStruct((NUM_CORES, n_pad), jnp.float32),
      scratch_types=[
          [pltpu.VMEM((CHUNK,), jnp.int32)] * 4,
          [pltpu.VMEM((CHUNK,), jnp.int32)] * 4,
          [pltpu.VMEM((CHUNK,), jnp.float32)] * 4,
          pltpu.VMEM((tab_n,), jnp.float32),
          pltpu.VMEM_SHARED((tab_n,), jnp.float32),
          [pltpu.SemaphoreType.DMA] * 4,
          [pltpu.SemaphoreType.DMA] * 4,
      ],
  )
  def agg1_kernel(src_hbm, dst_hbm, a_hbm, out_hbm,
                  srcs, dsts, vals, a_v, table, lsems, ssems):
    c = lax.axis_index("c")
    s = lax.axis_index("s")

    pltpu.sync_copy(a_hbm.at[pl.ds(0, tab_n)], a_v)
    _fill(vals[0], CHUNK, 0.0)
    _zero_slice_via(vals[0], table, s, zch)
    plsc.subcore_barrier()

    base0 = (c * NUM_SUBCORES + s) * per_tile
    _agg_pipeline(src_hbm, dst_hbm, a_v, table, base0, nchunks,
                  srcs, dsts, vals, lsems, ssems)
    plsc.subcore_barrier()

    @pl.when(s == 0)
    def _():
      pltpu.sync_copy(table, out_hbm.at[c, pl.ds(0, tab_n)])

  return agg1_kernel


def _make_agg2_kernel(e_pad, n_pad, tab_n):
  per_tile = e_pad // NUM_TILES
  nchunks = per_tile // CHUNK
  zch = tab_n // NUM_SUBCORES

  @functools.partial(
      pl.kernel,
      mesh=_mesh(),
      compiler_params=pltpu.CompilerParams(needs_layout_passes=False),
      out_type=jax.ShapeDtypeStruct((NUM_CORES, 2, n_pad), jnp.float32),
      scratch_types=[
          [pltpu.VMEM((CHUNK,), jnp.int32)] * 4,
          [pltpu.VMEM((CHUNK,), jnp.int32)] * 4,
          [pltpu.VMEM((CHUNK,), jnp.float32)] * 4,
          pltpu.VMEM((tab_n,), jnp.float32),
          pltpu.VMEM_SHARED((tab_n,), jnp.float32),
          [pltpu.SemaphoreType.DMA] * 4,
          [pltpu.SemaphoreType.DMA] * 4,
      ],
  )
  def agg2_kernel(src_hbm, dst_hbm, g0_hbm, g1_hbm, out_hbm,
                  srcs, dsts, vals, g_v, table, lsems, ssems):
    c = lax.axis_index("c")
    s = lax.axis_index("s")

    base0 = (c * NUM_SUBCORES + s) * per_tile

    for f, g_hbm in enumerate((g0_hbm, g1_hbm)):
      pltpu.sync_copy(g_hbm.at[pl.ds(0, tab_n)], g_v)
      _fill(vals[0], CHUNK, 0.0)
      _zero_slice_via(vals[0], table, s, zch)
      plsc.subcore_barrier()

      _agg_pipeline(src_hbm, dst_hbm, g_v, table, base0, nchunks,
                    srcs, dsts, vals, lsems, ssems)
      plsc.subcore_barrier()

      @pl.when(s == 0)
      def _():
        pltpu.sync_copy(table, out_hbm.at[c, f, pl.ds(0, tab_n)])

      plsc.subcore_barrier()

  return agg2_kernel


def _tc1_body(degp_ref, x_ref, dinv_ref, a_ref):
  deg = degp_ref[0] + degp_ref[1] + 1.0
  dinv = lax.rsqrt(deg)
  dinv_ref[...] = dinv
  a_ref[...] = dinv * x_ref[...]


def _tc2_body(t1p_ref, a_ref, dinv_ref, w1_ref, b1_ref, w2_ref, b2_ref,
              g0_ref, g1_ref):
  del b2_ref
  dinv = dinv_ref[...]
  t1 = t1p_ref[0] + t1p_ref[1] + a_ref[...]
  sval = dinv * t1
  p0 = jnp.zeros_like(sval)
  p1 = jnp.zeros_like(sval)
  for j in range(32):
    hj = jnp.maximum(sval * w1_ref[0, j] + b1_ref[0, j], 0.0)
    p0 = p0 + hj * w2_ref[j, 0]
    p1 = p1 + hj * w2_ref[j, 1]
  g0_ref[...] = dinv * p0
  g1_ref[...] = dinv * p1


def _tc3_body(t2p_ref, g0_ref, g1_ref, dinv_ref, b2_ref, o0_ref, o1_ref):
  dinv = dinv_ref[...]
  z0 = dinv * (t2p_ref[0, 0] + t2p_ref[1, 0] + g0_ref[...]) + b2_ref[0, 0]
  z1 = dinv * (t2p_ref[0, 1] + t2p_ref[1, 1] + g1_ref[...]) + b2_ref[0, 1]
  m = jnp.maximum(z0, z1)
  lse = m + jnp.log(jnp.exp(z0 - m) + jnp.exp(z1 - m))
  o0_ref[...] = z0 - lse
  o1_ref[...] = z1 - lse


def kernel(x, edge_index, W1, b1, W2, b2):
  n = x.shape[0]
  e = edge_index.shape[1]

  n_pad = ((n + 1023) // 1024) * 1024
  tab_n = ((n + 127) // 128) * 128
  rows = n_pad // 128
  e_unit = NUM_TILES * CHUNK * 2
  e_pad = ((e + e_unit - 1) // e_unit) * e_unit

  src = edge_index[0]
  dst = edge_index[1]
  if e_pad != e:
    src = jnp.concatenate([src, jnp.full((e_pad - e,), n, jnp.int32)])
    dst = jnp.concatenate([dst, jnp.full((e_pad - e,), n, jnp.int32)])

  x_flat = jnp.pad(x[:, 0], (0, n_pad - n))

  deg_p = _make_deg_kernel(e_pad, n_pad)(dst)

  degp_r = deg_p.reshape(NUM_CORES, rows, 128)
  x_r = x_flat.reshape(rows, 128)
  dinv_r, a_r = pl.pallas_call(
      _tc1_body,
      out_shape=[
          jax.ShapeDtypeStruct((rows, 128), jnp.float32),
          jax.ShapeDtypeStruct((rows, 128), jnp.float32),
      ],
  )(degp_r, x_r)

  t1_p = _make_agg1_kernel(e_pad, n_pad, tab_n)(src, dst, a_r.reshape(n_pad))

  t1p_r = t1_p.reshape(NUM_CORES, rows, 128)
  g0_r, g1_r = pl.pallas_call(
      _tc2_body,
      out_shape=[
          jax.ShapeDtypeStruct((rows, 128), jnp.float32),
          jax.ShapeDtypeStruct((rows, 128), jnp.float32),
      ],
  )(t1p_r, a_r, dinv_r, W1.reshape(1, 32), b1.reshape(1, 32),
    W2.reshape(32, 2), b2.reshape(1, 2))

  t2_p = _make_agg2_kernel(e_pad, n_pad, tab_n)(
      src, dst, g0_r.reshape(n_pad), g1_r.reshape(n_pad))

  t2p_r = t2_p.reshape(NUM_CORES, 2, rows, 128)
  o0_r, o1_r = pl.pallas_call(
      _tc3_body,
      out_shape=[
          jax.ShapeDtypeStruct((rows, 128), jnp.float32),
          jax.ShapeDtypeStruct((rows, 128), jnp.float32),
      ],
  )(t2p_r, g0_r, g1_r, dinv_r, b2.reshape(1, 2))

  return jnp.stack([o0_r.reshape(n_pad)[:n], o1_r.reshape(n_pad)[:n]], axis=1)


# speedup vs baseline: 417.3676x; 1.4116x over previous
import functools

import jax
import jax.numpy as jnp
from jax import lax
from jax.experimental import pallas as pl
from jax.experimental.pallas import tpu as pltpu
from jax.experimental.pallas import tpu_sc as plsc

CHUNK = 2000
NUM_CORES = 2
NUM_SUBCORES = 16
NUM_TILES = NUM_CORES * NUM_SUBCORES


def _mesh():
  return plsc.VectorSubcoreMesh(core_axis_name="c", subcore_axis_name="s")


def _fill(ref, size, value):
  vec = jnp.full((16,), value, jnp.float32)

  @pl.loop(0, size // 16)
  def _(i):
    ref[pl.ds(i * 16, 16)] = vec


def _zero_my_slice(zeros_v, table, s, zch):
  pltpu.sync_copy(zeros_v, table.at[pl.ds(s * zch, zch)])


def _zero_slice_via(buf, table, s, zch):
  off = 0
  while off < zch:
    ln = min(CHUNK, zch - off)
    pltpu.sync_copy(buf.at[pl.ds(0, ln)], table.at[pl.ds(s * zch + off, ln)])
    off += ln


def _make_deg_kernel(e_pad, n_pad):
  per_tile = e_pad // NUM_TILES
  pairs = per_tile // CHUNK // 2
  zch = n_pad // NUM_SUBCORES

  @functools.partial(
      pl.kernel,
      mesh=_mesh(),
      compiler_params=pltpu.CompilerParams(needs_layout_passes=False),
      out_type=jax.ShapeDtypeStruct((NUM_CORES, n_pad), jnp.float32),
      scratch_types=[
          pltpu.VMEM((CHUNK,), jnp.int32),
          pltpu.VMEM((CHUNK,), jnp.int32),
          pltpu.VMEM((CHUNK,), jnp.float32),
          pltpu.VMEM((zch,), jnp.float32),
          pltpu.VMEM_SHARED((n_pad,), jnp.float32),
          pltpu.SemaphoreType.DMA,
          pltpu.SemaphoreType.DMA,
      ],
  )
  def deg_kernel(dst_hbm, out_hbm, dst0, dst1, ones_v, zeros_v, table,
                 sem0, sem1):
    c = lax.axis_index("c")
    s = lax.axis_index("s")
    _fill(zeros_v, zch, 0.0)
    _fill(ones_v, CHUNK, 1.0)
    _zero_my_slice(zeros_v, table, s, zch)
    plsc.subcore_barrier()

    base0 = (c * NUM_SUBCORES + s) * per_tile

    @pl.loop(0, pairs)
    def _(k):
      @pl.when(k > 0)
      def _():
        pltpu.make_async_copy(ones_v, table.at[dst0], sem0).wait()

      pltpu.sync_copy(dst_hbm.at[pl.ds(base0 + 2 * k * CHUNK, CHUNK)], dst0)
      pltpu.async_copy(ones_v, table.at[dst0], sem0, add=True)

      @pl.when(k > 0)
      def _():
        pltpu.make_async_copy(ones_v, table.at[dst1], sem1).wait()

      pltpu.sync_copy(
          dst_hbm.at[pl.ds(base0 + (2 * k + 1) * CHUNK, CHUNK)], dst1)
      pltpu.async_copy(ones_v, table.at[dst1], sem1, add=True)

    pltpu.make_async_copy(ones_v, table.at[dst0], sem0).wait()
    pltpu.make_async_copy(ones_v, table.at[dst1], sem1).wait()
    plsc.subcore_barrier()

    @pl.when(s == 0)
    def _():
      pltpu.sync_copy(table, out_hbm.at[c])

  return deg_kernel


def _agg_pipeline(src_hbm, dst_hbm, tab_v, table, base0, nchunks,
                  srcs, dsts, vals, lsems, ssems):
  quads = nchunks // 4

  def gather(src_v, vals_v):
    assert CHUNK % 80 == 0
    @pl.loop(0, CHUNK // 80)
    def _(j):
      b = j * 80
      idx = [src_v[pl.ds(b + 16 * u, 16)] for u in range(5)]
      got = [plsc.load_gather(tab_v, [i]) for i in idx]
      for u in range(5):
        vals_v[pl.ds(b + 16 * u, 16)] = got[u]

  def fire_loads(chunk, j):
    base = base0 + chunk * CHUNK
    pltpu.async_copy(src_hbm.at[pl.ds(base, CHUNK)], srcs[j], lsems[j])
    pltpu.async_copy(dst_hbm.at[pl.ds(base, CHUNK)], dsts[j], lsems[j])

  def wait_loads(j):
    pltpu.make_async_copy(src_hbm.at[pl.ds(0, CHUNK)], srcs[j],
                          lsems[j]).wait()
    pltpu.make_async_copy(dst_hbm.at[pl.ds(0, CHUNK)], dsts[j],
                          lsems[j]).wait()

  def wait_scat(j):
    pltpu.make_async_copy(vals[j], table.at[dsts[j]], ssems[j]).wait()

  fire_loads(0, 0)
  fire_loads(1, 1)

  @pl.loop(0, quads)
  def _(k):
    for j in range(4):
      jp = (j + 2) % 4
      wait_loads(j)
      if j < 2:
        @pl.when(k > 0)
        def _():
          wait_scat(jp)

        fire_loads(4 * k + j + 2, jp)
      else:
        wait_scat(jp)

        @pl.when(k < quads - 1)
        def _():
          fire_loads(4 * k + j + 2, jp)

      gather(srcs[j], vals[j])
      pltpu.async_copy(vals[j], table.at[dsts[j]], ssems[j], add=True)

  wait_scat(2)
  wait_scat(3)


def _make_agg1_kernel(e_pad, n_pad, tab_n):
  per_tile = e_pad // NUM_TILES
  nchunks = per_tile // CHUNK
  zch = tab_n // NUM_SUBCORES

  @functools.partial(
      pl.kernel,
      mesh=_mesh(),
      compiler_params=pltpu.CompilerParams(needs_layout_passes=False),
      out_type=jax.ShapeDtypeStruct((NUM_CORES, n_pad), jnp.float32),
      scratch_types=[
          [pltpu.VMEM((CHUNK,), jnp.int32)] * 4,
          [pltpu.VMEM((CHUNK,), jnp.int32)] * 4,
          [pltpu.VMEM((CHUNK,), jnp.float32)] * 4,
          pltpu.VMEM((tab_n,), jnp.float32),
          pltpu.VMEM_SHARED((tab_n,), jnp.float32),
          [pltpu.SemaphoreType.DMA] * 4,
          [pltpu.SemaphoreType.DMA] * 4,
      ],
  )
  def agg1_kernel(src_hbm, dst_hbm, a_hbm, out_hbm,
                  srcs, dsts, vals, a_v, table, lsems, ssems):
    c = lax.axis_index("c")
    s = lax.axis_index("s")

    pltpu.sync_copy(a_hbm.at[pl.ds(0, tab_n)], a_v)
    _fill(vals[0], CHUNK, 0.0)
    _zero_slice_via(vals[0], table, s, zch)
    plsc.subcore_barrier()

    base0 = (c * NUM_SUBCORES + s) * per_tile
    _agg_pipeline(src_hbm, dst_hbm, a_v, table, base0, nchunks,
                  srcs, dsts, vals, lsems, ssems)
    plsc.subcore_barrier()

    @pl.when(s == 0)
    def _():
      pltpu.sync_copy(table, out_hbm.at[c, pl.ds(0, tab_n)])

  return agg1_kernel


def _make_agg2_kernel(e_pad, n_pad, tab_n):
  per_tile = e_pad // NUM_TILES
  nchunks = per_tile // CHUNK
  zch = tab_n // NUM_SUBCORES

  @functools.partial(
      pl.kernel,
      mesh=_mesh(),
      compiler_params=pltpu.CompilerParams(needs_layout_passes=False),
      out_type=jax.ShapeDtypeStruct((NUM_CORES, 2, n_pad), jnp.float32),
      scratch_types=[
          [pltpu.VMEM((CHUNK,), jnp.int32)] * 4,
          [pltpu.VMEM((CHUNK,), jnp.int32)] * 4,
          [pltpu.VMEM((CHUNK,), jnp.float32)] * 4,
          pltpu.VMEM((tab_n,), jnp.float32),
          pltpu.VMEM_SHARED((tab_n,), jnp.float32),
          [pltpu.SemaphoreType.DMA] * 4,
          [pltpu.SemaphoreType.DMA] * 4,
      ],
  )
  def agg2_kernel(src_hbm, dst_hbm, g0_hbm, g1_hbm, out_hbm,
                  srcs, dsts, vals, g_v, table, lsems, ssems):
    c = lax.axis_index("c")
    s = lax.axis_index("s")

    base0 = (c * NUM_SUBCORES + s) * per_tile

    for f, g_hbm in enumerate((g0_hbm, g1_hbm)):
      pltpu.sync_copy(g_hbm.at[pl.ds(0, tab_n)], g_v)
      _fill(vals[0], CHUNK, 0.0)
      _zero_slice_via(vals[0], table, s, zch)
      plsc.subcore_barrier()

      _agg_pipeline(src_hbm, dst_hbm, g_v, table, base0, nchunks,
                    srcs, dsts, vals, lsems, ssems)
      plsc.subcore_barrier()

      @pl.when(s == 0)
      def _():
        pltpu.sync_copy(table, out_hbm.at[c, f, pl.ds(0, tab_n)])

      plsc.subcore_barrier()

  return agg2_kernel


def _tc1_body(degp_ref, x_ref, dinv_ref, a_ref):
  deg = degp_ref[0] + degp_ref[1] + 1.0
  dinv = lax.rsqrt(deg)
  dinv_ref[...] = dinv
  a_ref[...] = dinv * x_ref[...]


def _tc2_body(t1p_ref, a_ref, dinv_ref, w1_ref, b1_ref, w2_ref, b2_ref,
              g0_ref, g1_ref):
  del b2_ref
  dinv = dinv_ref[...]
  t1 = t1p_ref[0] + t1p_ref[1] + a_ref[...]
  sval = dinv * t1
  p0 = jnp.zeros_like(sval)
  p1 = jnp.zeros_like(sval)
  for j in range(32):
    hj = jnp.maximum(sval * w1_ref[0, j] + b1_ref[0, j], 0.0)
    p0 = p0 + hj * w2_ref[j, 0]
    p1 = p1 + hj * w2_ref[j, 1]
  g0_ref[...] = dinv * p0
  g1_ref[...] = dinv * p1


def _tc3_body(t2p_ref, g0_ref, g1_ref, dinv_ref, b2_ref, o0_ref, o1_ref):
  dinv = dinv_ref[...]
  z0 = dinv * (t2p_ref[0, 0] + t2p_ref[1, 0] + g0_ref[...]) + b2_ref[0, 0]
  z1 = dinv * (t2p_ref[0, 1] + t2p_ref[1, 1] + g1_ref[...]) + b2_ref[0, 1]
  m = jnp.maximum(z0, z1)
  lse = m + jnp.log(jnp.exp(z0 - m) + jnp.exp(z1 - m))
  o0_ref[...] = z0 - lse
  o1_ref[...] = z1 - lse


def kernel(x, edge_index, W1, b1, W2, b2):
  n = x.shape[0]
  e = edge_index.shape[1]

  n_pad = ((n + 1023) // 1024) * 1024
  tab_n = ((n + 127) // 128) * 128
  rows = n_pad // 128
  e_unit = NUM_TILES * CHUNK * 2
  e_pad = ((e + e_unit - 1) // e_unit) * e_unit

  src = edge_index[0]
  dst = edge_index[1]
  if e_pad != e:
    src = jnp.concatenate([src, jnp.full((e_pad - e,), n, jnp.int32)])
    dst = jnp.concatenate([dst, jnp.full((e_pad - e,), n, jnp.int32)])

  x_flat = jnp.pad(x[:, 0], (0, n_pad - n))

  deg_p = _make_deg_kernel(e_pad, n_pad)(dst)

  degp_r = deg_p.reshape(NUM_CORES, rows, 128)
  x_r = x_flat.reshape(rows, 128)
  dinv_r, a_r = pl.pallas_call(
      _tc1_body,
      out_shape=[
          jax.ShapeDtypeStruct((rows, 128), jnp.float32),
          jax.ShapeDtypeStruct((rows, 128), jnp.float32),
      ],
  )(degp_r, x_r)

  t1_p = _make_agg1_kernel(e_pad, n_pad, tab_n)(src, dst, a_r.reshape(n_pad))

  t1p_r = t1_p.reshape(NUM_CORES, rows, 128)
  g0_r, g1_r = pl.pallas_call(
      _tc2_body,
      out_shape=[
          jax.ShapeDtypeStruct((rows, 128), jnp.float32),
          jax.ShapeDtypeStruct((rows, 128), jnp.float32),
      ],
  )(t1p_r, a_r, dinv_r, W1.reshape(1, 32), b1.reshape(1, 32),
    W2.reshape(32, 2), b2.reshape(1, 2))

  t2_p = _make_agg2_kernel(e_pad, n_pad, tab_n)(
      src, dst, g0_r.reshape(n_pad), g1_r.reshape(n_pad))

  t2p_r = t2_p.reshape(NUM_CORES, 2, rows, 128)
  o0_r, o1_r = pl.pallas_call(
      _tc3_body,
      out_shape=[
          jax.ShapeDtypeStruct((rows, 128), jnp.float32),
          jax.ShapeDtypeStruct((rows, 128), jnp.float32),
      ],
  )(t2p_r, g0_r, g1_r, dinv_r, b2.reshape(1, 2))

  return jnp.stack([o0_r.reshape(n_pad)[:n], o1_r.reshape(n_pad)[:n]], axis=1)
